# Initial kernel scaffold; baseline (speedup 1.0000x reference)
import functools

import jax
import jax.numpy as jnp
import numpy as np
from jax.experimental import pallas as pl
from jax.experimental.pallas import tpu as pltpu

N = 2048
IN_F = 512
OUT_F = 128
NF = 4
ORDER = 8
TOPK = 5
LMAX = 2.0

HI = jax.lax.Precision.HIGHEST


def _coeff_body(pts_ref, w1_ref, b1_ref, w2_ref, b2_ref, w3_ref, b3_ref,
                w4_ref, b4_ref, cosm_ref, c_ref):
    pts = pts_ref[...]
    h1 = jnp.maximum(pts * w1_ref[...] + b1_ref[...], 0.0)
    h2 = jnp.maximum(jnp.dot(h1, w2_ref[...], precision=HI) + b2_ref[...], 0.0)
    h3 = jnp.maximum(jnp.dot(h2, w3_ref[...], precision=HI) + b3_ref[...], 0.0)
    g = jnp.maximum(jnp.dot(h3, w4_ref[...], precision=HI) + b4_ref[...], 0.0)
    npts = ORDER + 1
    acc = jnp.zeros((16, 128), jnp.float32)
    for n in range(npts):
        acc = acc + cosm_ref[:, n:n + 1] * g[n:n + 1, :]
    c_ref[...] = (2.0 / npts) * acc


def _coeffs(W1, b1, W2, b2, W3, b3, W4, b4):
    npts = ORDER + 1
    a = LMAX / 2.0
    n = np.arange(npts, dtype=np.float32)
    pts = (a * np.cos(np.pi * (n + 0.5) / npts) + a).astype(np.float32)
    pts_pad = np.zeros((16, 1), np.float32)
    pts_pad[:npts, 0] = pts
    k = np.arange(npts, dtype=np.float32)
    cosm = np.cos(np.pi * k[:, None] * (n[None, :] + 0.5) / npts)
    cosm_pad = np.zeros((16, 128), np.float32)
    cosm_pad[:npts, :npts] = cosm

    def row(v, w):
        return jnp.zeros((1, w), jnp.float32).at[0, :v.shape[0]].set(v)

    def mat(m):
        t = m.T
        return jnp.zeros((128, 128), jnp.float32).at[:t.shape[0], :t.shape[1]].set(t)

    c_pad = pl.pallas_call(
        _coeff_body,
        out_shape=jax.ShapeDtypeStruct((16, 128), jnp.float32),
    )(jnp.asarray(pts_pad), row(W1[:, 0], 128), row(b1, 128),
      mat(W2), row(b2, 128), mat(W3), row(b3, 128), mat(W4), row(b4, 128),
      jnp.asarray(cosm_pad))
    return c_pad[:npts, :NF]


def _h_body(x_ref, w_ref, o_ref):
    o_ref[...] = jnp.dot(x_ref[...], w_ref[...], precision=HI)


def _linear(x, lin_W):
    bj = 256
    return pl.pallas_call(
        _h_body,
        grid=(N // bj,),
        in_specs=[
            pl.BlockSpec((bj, IN_F), lambda j: (j, 0)),
            pl.BlockSpec((IN_F, OUT_F), lambda j: (0, 0)),
        ],
        out_specs=pl.BlockSpec((bj, OUT_F), lambda j: (j, 0)),
        out_shape=jax.ShapeDtypeStruct((N, OUT_F), jnp.float32),
    )(x, lin_W.T)


def _dinv_body(a_ref, o_ref):
    deg = jnp.sum(a_ref[...], axis=1)
    o_ref[...] = jnp.where(deg > 0.0, 1.0 / jnp.sqrt(deg), 0.0)[None, :]


def _dinv(A):
    bj = 128
    d2 = pl.pallas_call(
        _dinv_body,
        grid=(N // bj,),
        in_specs=[pl.BlockSpec((bj, N), lambda j: (j, 0))],
        out_specs=pl.BlockSpec((1, bj), lambda j: (j, 0)),
        out_shape=jax.ShapeDtypeStruct((N // bj, bj), jnp.float32),
    )(A)
    return d2.reshape(N)


BJ = 256


def _main_body(a_ref, drow_ref, dcol_ref, h_ref, c_ref, att_ref, out_ref):
    j = pl.program_id(0)
    drow = drow_ref[...]
    dcol = dcol_ref[...]
    a_blk = a_ref[pl.ds(j * BJ, BJ), :]

    rows = j * BJ + jax.lax.broadcasted_iota(jnp.int32, (BJ, N), 0)
    cols = jax.lax.broadcasted_iota(jnp.int32, (BJ, N), 1)

    r0 = jnp.where(rows == cols, 1.0, 0.0)
    r1 = -(dcol * a_blk * drow)

    accs = [0.5 * c_ref[0, f] * r0 + c_ref[1, f] * r1 for f in range(NF)]
    rkm2, rkm1 = r0, r1
    for k in range(2, ORDER + 1):
        t = jnp.dot(rkm1 * drow, a_ref[...], precision=HI)
        rk = -2.0 * (t * drow) - rkm2
        for f in range(NF):
            accs[f] = accs[f] + c_ref[k, f] * rk
        rkm2, rkm1 = rkm1, rk

    h = h_ref[...]
    for f in range(NF):
        acc = accs[f]
        work = acc
        sel = jnp.zeros((BJ, N), jnp.bool_)
        vals = []
        for _ in range(TOPK):
            m = jnp.max(work, axis=1, keepdims=True)
            eq = work == m
            pos = jnp.min(jnp.where(eq, cols, N), axis=1, keepdims=True)
            newsel = cols == pos
            sel = jnp.logical_or(sel, newsel)
            work = jnp.where(newsel, -3e38, work)
            vals.append(m)
        maxv = vals[0]
        den = vals[0] * 0.0
        for m in vals:
            den = den + jnp.exp(m - maxv)
        att_f = jnp.where(sel, jnp.exp(acc - maxv) / den, 0.0)
        att_ref[f, :, :] = att_f
        out_ref[:, f * OUT_F:(f + 1) * OUT_F] = jnp.dot(att_f, h, precision=HI)


def _main(A, dinv, h, c):
    att4, out_nodes = pl.pallas_call(
        _main_body,
        grid=(N // BJ,),
        in_specs=[
            pl.BlockSpec((N, N), lambda j: (0, 0)),
            pl.BlockSpec((1, N), lambda j: (0, 0)),
            pl.BlockSpec((BJ, 1), lambda j: (j, 0)),
            pl.BlockSpec((N, OUT_F), lambda j: (0, 0)),
            pl.BlockSpec(memory_space=pltpu.SMEM),
        ],
        out_specs=[
            pl.BlockSpec((NF, BJ, N), lambda j: (0, j, 0)),
            pl.BlockSpec((BJ, NF * OUT_F), lambda j: (j, 0)),
        ],
        out_shape=[
            jax.ShapeDtypeStruct((NF, N, N), jnp.float32),
            jax.ShapeDtypeStruct((N, NF * OUT_F), jnp.float32),
        ],
    )(A, dinv.reshape(1, N), dinv.reshape(N, 1), h, c)
    return att4, out_nodes


def kernel(x, edge_index, lin_W, W1, b1, W2, b2, W3, b3, W4, b4):
    src, dst = edge_index[0], edge_index[1]
    A = jnp.zeros((N, N), jnp.float32).at[src, dst].set(1.0)
    A = jnp.maximum(A, A.T)

    c = _coeffs(W1, b1, W2, b2, W3, b3, W4, b4)
    h = _linear(x, lin_W)
    dinv = _dinv(A)
    att4, out_nodes = _main(A, dinv, h, c)
    return out_nodes, att4.reshape(NF * N, N)


# speedup vs baseline: 2.9142x; 2.9142x over previous
import jax
import jax.numpy as jnp
import numpy as np
from jax.experimental import pallas as pl
from jax.experimental.pallas import tpu as pltpu

N = 2048
IN_F = 512
OUT_F = 128
NF = 4
ORDER = 8
TOPK = 5
LMAX = 2.0

HI = jax.lax.Precision.HIGHEST
DEF = jax.lax.Precision.DEFAULT


def _coeff_body(pts_ref, w1_ref, b1_ref, w2_ref, b2_ref, w3_ref, b3_ref,
                w4_ref, b4_ref, cosm_ref, c_ref):
    pts = pts_ref[...]
    h1 = jnp.maximum(pts * w1_ref[...] + b1_ref[...], 0.0)
    h2 = jnp.maximum(jnp.dot(h1, w2_ref[...], precision=HI) + b2_ref[...], 0.0)
    h3 = jnp.maximum(jnp.dot(h2, w3_ref[...], precision=HI) + b3_ref[...], 0.0)
    g = jnp.maximum(jnp.dot(h3, w4_ref[...], precision=HI) + b4_ref[...], 0.0)
    npts = ORDER + 1
    acc = jnp.zeros((16, 128), jnp.float32)
    for n in range(npts):
        acc = acc + cosm_ref[:, n:n + 1] * g[n:n + 1, :]
    c_ref[...] = (2.0 / npts) * acc


def _coeffs(W1, b1, W2, b2, W3, b3, W4, b4):
    npts = ORDER + 1
    a = LMAX / 2.0
    n = np.arange(npts, dtype=np.float32)
    pts = (a * np.cos(np.pi * (n + 0.5) / npts) + a).astype(np.float32)
    pts_pad = np.zeros((16, 1), np.float32)
    pts_pad[:npts, 0] = pts
    k = np.arange(npts, dtype=np.float32)
    cosm = np.cos(np.pi * k[:, None] * (n[None, :] + 0.5) / npts)
    cosm_pad = np.zeros((16, 128), np.float32)
    cosm_pad[:npts, :npts] = cosm

    def row(v, w):
        return jnp.zeros((1, w), jnp.float32).at[0, :v.shape[0]].set(v)

    def mat(m):
        t = m.T
        return jnp.zeros((128, 128), jnp.float32).at[:t.shape[0], :t.shape[1]].set(t)

    c_pad = pl.pallas_call(
        _coeff_body,
        out_shape=jax.ShapeDtypeStruct((16, 128), jnp.float32),
    )(jnp.asarray(pts_pad), row(W1[:, 0], 128), row(b1, 128),
      mat(W2), row(b2, 128), mat(W3), row(b3, 128), mat(W4), row(b4, 128),
      jnp.asarray(cosm_pad))
    return c_pad[:npts, :NF]


def _h_body(x_ref, w_ref, o_ref):
    o_ref[...] = jnp.dot(x_ref[...], w_ref[...], precision=DEF)


def _linear(x, lin_W):
    bj = 256
    return pl.pallas_call(
        _h_body,
        grid=(N // bj,),
        in_specs=[
            pl.BlockSpec((bj, IN_F), lambda j: (j, 0)),
            pl.BlockSpec((IN_F, OUT_F), lambda j: (0, 0)),
        ],
        out_specs=pl.BlockSpec((bj, OUT_F), lambda j: (j, 0)),
        out_shape=jax.ShapeDtypeStruct((N, OUT_F), jnp.float32),
    )(x, lin_W.T)


def _dinv_body(a_ref, o_ref):
    deg = jnp.sum(a_ref[...], axis=1)
    o_ref[...] = jnp.where(deg > 0.0, 1.0 / jnp.sqrt(deg), 0.0)[None, None, :]


def _dinv(A):
    bj = 128
    d2 = pl.pallas_call(
        _dinv_body,
        grid=(N // bj,),
        in_specs=[pl.BlockSpec((bj, N), lambda j: (j, 0))],
        out_specs=pl.BlockSpec((1, 1, bj), lambda j: (j, 0, 0)),
        out_shape=jax.ShapeDtypeStruct((N // bj, 1, bj), jnp.float32),
    )(A)
    return d2.reshape(N)


def _lbuild_body(a_ref, drow_ref, dcol_ref, l_ref, t1_ref):
    j = pl.program_id(0)
    bj = a_ref.shape[0]
    rows = j * bj + jax.lax.broadcasted_iota(jnp.int32, (bj, N), 0)
    cols = jax.lax.broadcasted_iota(jnp.int32, (bj, N), 1)
    eye = jnp.where(rows == cols, 1.0, 0.0)
    s = (dcol_ref[...] * a_ref[...]) * drow_ref[...]
    l = eye - s
    l_ref[...] = l
    t1_ref[...] = l - eye


def _lbuild(A, dinv):
    bj = 256
    return pl.pallas_call(
        _lbuild_body,
        grid=(N // bj,),
        in_specs=[
            pl.BlockSpec((bj, N), lambda j: (j, 0)),
            pl.BlockSpec((1, N), lambda j: (0, 0)),
            pl.BlockSpec((bj, 1), lambda j: (j, 0)),
        ],
        out_specs=[
            pl.BlockSpec((bj, N), lambda j: (j, 0)),
            pl.BlockSpec((bj, N), lambda j: (j, 0)),
        ],
        out_shape=[
            jax.ShapeDtypeStruct((N, N), jnp.float32),
            jax.ShapeDtypeStruct((N, N), jnp.float32),
        ],
    )(A, dinv.reshape(1, N), dinv.reshape(N, 1))


BMM = 256


def _mm_body(l_ref, t1_ref, p_ref):
    p_ref[...] = jnp.dot(l_ref[...], t1_ref[...], precision=DEF)


def _cheb_mm(L, T1):
    return pl.pallas_call(
        _mm_body,
        grid=(N // BMM,),
        in_specs=[
            pl.BlockSpec((BMM, N), lambda j: (j, 0)),
            pl.BlockSpec((N, N), lambda j: (0, 0)),
        ],
        out_specs=pl.BlockSpec((BMM, N), lambda j: (j, 0)),
        out_shape=jax.ShapeDtypeStruct((N, N), jnp.float32),
    )(L, T1)


BJ = 128


def _final_body(l_ref, t2_ref, t3_ref, t4_ref, t5_ref, t6_ref, t7_ref,
                t8_ref, h_ref, c_ref, att_ref, out_ref):
    j = pl.program_id(0)
    rows = j * BJ + jax.lax.broadcasted_iota(jnp.int32, (BJ, N), 0)
    cols = jax.lax.broadcasted_iota(jnp.int32, (BJ, N), 1)
    eye = jnp.where(rows == cols, 1.0, 0.0)
    t1 = l_ref[...] - eye
    tk = [t2_ref[...], t3_ref[...], t4_ref[...], t5_ref[...], t6_ref[...],
          t7_ref[...], t8_ref[...]]
    h = h_ref[...]
    for f in range(NF):
        acc = (0.5 * c_ref[0, f]) * eye + c_ref[1, f] * t1
        for k in range(2, ORDER + 1):
            acc = acc + c_ref[k, f] * tk[k - 2]
        work = acc
        sel = jnp.zeros((BJ, N), jnp.bool_)
        vals = []
        for _ in range(TOPK):
            m = jnp.max(work, axis=1, keepdims=True)
            eq = work == m
            pos = jnp.min(jnp.where(eq, cols, N), axis=1, keepdims=True)
            newsel = cols == pos
            sel = jnp.logical_or(sel, newsel)
            work = jnp.where(newsel, -3e38, work)
            vals.append(m)
        maxv = vals[0]
        den = vals[0] * 0.0
        for m in vals:
            den = den + jnp.exp(m - maxv)
        att_f = jnp.where(sel, jnp.exp(acc - maxv) / den, 0.0)
        att_ref[f, :, :] = att_f
        out_ref[:, f * OUT_F:(f + 1) * OUT_F] = jnp.dot(att_f, h, precision=DEF)


def _finalize(L, Ts, h, c):
    blk = lambda: pl.BlockSpec((BJ, N), lambda j: (j, 0))
    att4, out_nodes = pl.pallas_call(
        _final_body,
        grid=(N // BJ,),
        in_specs=[blk() for _ in range(8)] + [
            pl.BlockSpec((N, OUT_F), lambda j: (0, 0)),
            pl.BlockSpec(memory_space=pltpu.SMEM),
        ],
        out_specs=[
            pl.BlockSpec((NF, BJ, N), lambda j: (0, j, 0)),
            pl.BlockSpec((BJ, NF * OUT_F), lambda j: (j, 0)),
        ],
        out_shape=[
            jax.ShapeDtypeStruct((NF, N, N), jnp.float32),
            jax.ShapeDtypeStruct((N, NF * OUT_F), jnp.float32),
        ],
    )(L, *Ts, h, c)
    return att4, out_nodes


def _mlp_c(lam, W1, b1, W2, b2, W3, b3, W4, b4):
    x = lam.reshape(-1, 1)
    x = jax.nn.relu(x @ W1.T + b1)
    x = jax.nn.relu(x @ W2.T + b2)
    x = jax.nn.relu(x @ W3.T + b3)
    g_pts = jax.nn.relu(x @ W4.T + b4)
    npts = ORDER + 1
    n = jnp.arange(npts, dtype=jnp.float32)
    k = jnp.arange(ORDER + 1, dtype=jnp.float32)
    cosm = jnp.cos(jnp.pi * k[:, None] * (n[None, :] + 0.5) / npts)
    return (2.0 / npts) * (cosm @ g_pts)


def kernel(x, edge_index, lin_W, W1, b1, W2, b2, W3, b3, W4, b4):
    src, dst = edge_index[0], edge_index[1]
    A = jnp.zeros((N, N), jnp.float32).at[src, dst].set(1.0)
    A = jnp.maximum(A, A.T)
    deg = A.sum(axis=1)
    dinv = jnp.where(deg > 0, 1.0 / jnp.sqrt(jnp.maximum(deg, 1e-12)), 0.0)
    I = jnp.eye(N, dtype=jnp.float32)
    L = I - dinv[:, None] * A * dinv[None, :]

    a1 = LMAX / 2.0
    a2 = LMAX / 2.0
    npts = ORDER + 1
    n = jnp.arange(npts, dtype=jnp.float32)
    pts = a1 * jnp.cos(jnp.pi * (n + 0.5) / npts) + a2
    c = _mlp_c(pts, W1, b1, W2, b2, W3, b3, W4, b4)

    h = _linear(x, lin_W)

    Ts = []
    t0, t1 = I, (L - a2 * I) / a1
    for _ in range(2, ORDER + 1):
        tn = (2.0 / a1) * (_cheb_mm(L, t1) - a2 * t1) - t0
        Ts.append(tn)
        t0, t1 = t1, tn

    att4, out_nodes = _finalize(L, Ts, h, c)
    return out_nodes, att4.reshape(NF * N, N)


# speedup vs baseline: 3.4747x; 1.1923x over previous
import jax
import jax.numpy as jnp
import numpy as np
from jax.experimental import pallas as pl
from jax.experimental.pallas import tpu as pltpu

N = 2048
IN_F = 512
OUT_F = 128
NF = 4
ORDER = 8
TOPK = 5
LMAX = 2.0

HI = jax.lax.Precision.HIGHEST
DEF = jax.lax.Precision.DEFAULT


def _coeff_body(pts_ref, w1_ref, b1_ref, w2_ref, b2_ref, w3_ref, b3_ref,
                w4_ref, b4_ref, cosm_ref, c_ref):
    pts = pts_ref[...]
    h1 = jnp.maximum(pts * w1_ref[...] + b1_ref[...], 0.0)
    h2 = jnp.maximum(jnp.dot(h1, w2_ref[...], precision=HI) + b2_ref[...], 0.0)
    h3 = jnp.maximum(jnp.dot(h2, w3_ref[...], precision=HI) + b3_ref[...], 0.0)
    g = jnp.maximum(jnp.dot(h3, w4_ref[...], precision=HI) + b4_ref[...], 0.0)
    npts = ORDER + 1
    acc = jnp.zeros((16, 128), jnp.float32)
    for n in range(npts):
        acc = acc + cosm_ref[:, n:n + 1] * g[n:n + 1, :]
    c_ref[...] = (2.0 / npts) * acc


def _coeffs(W1, b1, W2, b2, W3, b3, W4, b4):
    npts = ORDER + 1
    a = LMAX / 2.0
    n = np.arange(npts, dtype=np.float32)
    pts = (a * np.cos(np.pi * (n + 0.5) / npts) + a).astype(np.float32)
    pts_pad = np.zeros((16, 1), np.float32)
    pts_pad[:npts, 0] = pts
    k = np.arange(npts, dtype=np.float32)
    cosm = np.cos(np.pi * k[:, None] * (n[None, :] + 0.5) / npts)
    cosm_pad = np.zeros((16, 128), np.float32)
    cosm_pad[:npts, :npts] = cosm

    def row(v, w):
        return jnp.zeros((1, w), jnp.float32).at[0, :v.shape[0]].set(v)

    def mat(m):
        t = m.T
        return jnp.zeros((128, 128), jnp.float32).at[:t.shape[0], :t.shape[1]].set(t)

    c_pad = pl.pallas_call(
        _coeff_body,
        out_shape=jax.ShapeDtypeStruct((16, 128), jnp.float32),
    )(jnp.asarray(pts_pad), row(W1[:, 0], 128), row(b1, 128),
      mat(W2), row(b2, 128), mat(W3), row(b3, 128), mat(W4), row(b4, 128),
      jnp.asarray(cosm_pad))
    return c_pad[:npts, :NF]


def _h_body(x_ref, w_ref, o_ref):
    o_ref[...] = jnp.dot(x_ref[...], w_ref[...], precision=DEF)


def _linear(x, lin_W):
    bj = 256
    return pl.pallas_call(
        _h_body,
        grid=(N // bj,),
        in_specs=[
            pl.BlockSpec((bj, IN_F), lambda j: (j, 0)),
            pl.BlockSpec((IN_F, OUT_F), lambda j: (0, 0)),
        ],
        out_specs=pl.BlockSpec((bj, OUT_F), lambda j: (j, 0)),
        out_shape=jax.ShapeDtypeStruct((N, OUT_F), jnp.float32),
    )(x, lin_W.T)


def _dinv_body(a_ref, o_ref):
    deg = jnp.sum(a_ref[...], axis=1)
    o_ref[...] = jnp.where(deg > 0.0, 1.0 / jnp.sqrt(deg), 0.0)[None, None, :]


def _dinv(A):
    bj = 128
    d2 = pl.pallas_call(
        _dinv_body,
        grid=(N // bj,),
        in_specs=[pl.BlockSpec((bj, N), lambda j: (j, 0))],
        out_specs=pl.BlockSpec((1, 1, bj), lambda j: (j, 0, 0)),
        out_shape=jax.ShapeDtypeStruct((N // bj, 1, bj), jnp.float32),
    )(A)
    return d2.reshape(N)


def _lbuild_body(a_ref, drow_ref, dcol_ref, l_ref, t1_ref):
    j = pl.program_id(0)
    bj = a_ref.shape[0]
    rows = j * bj + jax.lax.broadcasted_iota(jnp.int32, (bj, N), 0)
    cols = jax.lax.broadcasted_iota(jnp.int32, (bj, N), 1)
    eye = jnp.where(rows == cols, 1.0, 0.0)
    s = (dcol_ref[...] * a_ref[...]) * drow_ref[...]
    l = eye - s
    l_ref[...] = l
    t1_ref[...] = l - eye


def _lbuild(A, dinv):
    bj = 256
    return pl.pallas_call(
        _lbuild_body,
        grid=(N // bj,),
        in_specs=[
            pl.BlockSpec((bj, N), lambda j: (j, 0)),
            pl.BlockSpec((1, N), lambda j: (0, 0)),
            pl.BlockSpec((bj, 1), lambda j: (j, 0)),
        ],
        out_specs=[
            pl.BlockSpec((bj, N), lambda j: (j, 0)),
            pl.BlockSpec((bj, N), lambda j: (j, 0)),
        ],
        out_shape=[
            jax.ShapeDtypeStruct((N, N), jnp.float32),
            jax.ShapeDtypeStruct((N, N), jnp.float32),
        ],
    )(A, dinv.reshape(1, N), dinv.reshape(N, 1))


BMM = 256


def _mm_body(l_ref, t1_ref, p_ref):
    p_ref[...] = jnp.dot(l_ref[...], t1_ref[...], precision=DEF)


def _cheb_mm(L, T1):
    return pl.pallas_call(
        _mm_body,
        grid=(N // BMM,),
        in_specs=[
            pl.BlockSpec((BMM, N), lambda j: (j, 0)),
            pl.BlockSpec((N, N), lambda j: (0, 0)),
        ],
        out_specs=pl.BlockSpec((BMM, N), lambda j: (j, 0)),
        out_shape=jax.ShapeDtypeStruct((N, N), jnp.float32),
    )(L, T1)


def _cheb_fused_body(l_ref, t1_ref, t0_ref, tn_ref):
    j = pl.program_id(0)
    p = jnp.dot(l_ref[...], t1_ref[...], precision=DEF)
    t1_blk = t1_ref[pl.ds(j * BMM, BMM), :]
    tn_ref[...] = 2.0 * (p - t1_blk) - t0_ref[...]


def _cheb_step_fused(L, T1, T0):
    return pl.pallas_call(
        _cheb_fused_body,
        grid=(N // BMM,),
        in_specs=[
            pl.BlockSpec((BMM, N), lambda j: (j, 0)),
            pl.BlockSpec((N, N), lambda j: (0, 0)),
            pl.BlockSpec((BMM, N), lambda j: (j, 0)),
        ],
        out_specs=pl.BlockSpec((BMM, N), lambda j: (j, 0)),
        out_shape=jax.ShapeDtypeStruct((N, N), jnp.float32),
    )(L, T1, T0)


BJ = 128


def _final_body(l_ref, t2_ref, t3_ref, t4_ref, t5_ref, t6_ref, t7_ref,
                t8_ref, h_ref, c_ref, att_ref, out_ref):
    j = pl.program_id(0)
    rows = j * BJ + jax.lax.broadcasted_iota(jnp.int32, (BJ, N), 0)
    cols = jax.lax.broadcasted_iota(jnp.int32, (BJ, N), 1)
    eye = jnp.where(rows == cols, 1.0, 0.0)
    t1 = l_ref[...] - eye
    tk = [t2_ref[...], t3_ref[...], t4_ref[...], t5_ref[...], t6_ref[...],
          t7_ref[...], t8_ref[...]]
    h = h_ref[...]
    for f in range(NF):
        acc = (0.5 * c_ref[0, f]) * eye + c_ref[1, f] * t1
        for k in range(2, ORDER + 1):
            acc = acc + c_ref[k, f] * tk[k - 2]
        work = acc
        sel = jnp.zeros((BJ, N), jnp.bool_)
        vals = []
        for _ in range(TOPK):
            m = jnp.max(work, axis=1, keepdims=True)
            eq = work == m
            pos = jnp.min(jnp.where(eq, cols, N), axis=1, keepdims=True)
            newsel = cols == pos
            sel = jnp.logical_or(sel, newsel)
            work = jnp.where(newsel, -3e38, work)
            vals.append(m)
        maxv = vals[0]
        den = vals[0] * 0.0
        for m in vals:
            den = den + jnp.exp(m - maxv)
        att_f = jnp.where(sel, jnp.exp(acc - maxv) / den, 0.0)
        att_ref[f, :, :] = att_f
        out_ref[:, f * OUT_F:(f + 1) * OUT_F] = jnp.dot(att_f, h, precision=DEF)


def _finalize(L, Ts, h, c):
    blk = lambda: pl.BlockSpec((BJ, N), lambda j: (j, 0))
    att4, out_nodes = pl.pallas_call(
        _final_body,
        grid=(N // BJ,),
        in_specs=[blk() for _ in range(8)] + [
            pl.BlockSpec((N, OUT_F), lambda j: (0, 0)),
            pl.BlockSpec(memory_space=pltpu.SMEM),
        ],
        out_specs=[
            pl.BlockSpec((NF, BJ, N), lambda j: (0, j, 0)),
            pl.BlockSpec((BJ, NF * OUT_F), lambda j: (j, 0)),
        ],
        out_shape=[
            jax.ShapeDtypeStruct((NF, N, N), jnp.float32),
            jax.ShapeDtypeStruct((N, NF * OUT_F), jnp.float32),
        ],
    )(L, *Ts, h, c)
    return att4, out_nodes


def _mlp_c(lam, W1, b1, W2, b2, W3, b3, W4, b4):
    x = lam.reshape(-1, 1)
    x = jax.nn.relu(x @ W1.T + b1)
    x = jax.nn.relu(x @ W2.T + b2)
    x = jax.nn.relu(x @ W3.T + b3)
    g_pts = jax.nn.relu(x @ W4.T + b4)
    npts = ORDER + 1
    n = jnp.arange(npts, dtype=jnp.float32)
    k = jnp.arange(ORDER + 1, dtype=jnp.float32)
    cosm = jnp.cos(jnp.pi * k[:, None] * (n[None, :] + 0.5) / npts)
    return (2.0 / npts) * (cosm @ g_pts)


def kernel(x, edge_index, lin_W, W1, b1, W2, b2, W3, b3, W4, b4):
    src, dst = edge_index[0], edge_index[1]
    A = jnp.zeros((N, N), jnp.float32).at[src, dst].set(1.0)
    A = jnp.maximum(A, A.T)
    deg = A.sum(axis=1)
    dinv = jnp.where(deg > 0, 1.0 / jnp.sqrt(jnp.maximum(deg, 1e-12)), 0.0)
    I = jnp.eye(N, dtype=jnp.float32)
    L = I - dinv[:, None] * A * dinv[None, :]

    a1 = LMAX / 2.0
    a2 = LMAX / 2.0
    npts = ORDER + 1
    n = jnp.arange(npts, dtype=jnp.float32)
    pts = a1 * jnp.cos(jnp.pi * (n + 0.5) / npts) + a2
    c = _mlp_c(pts, W1, b1, W2, b2, W3, b3, W4, b4)

    h = _linear(x, lin_W)

    Ts = []
    t0, t1 = I, (L - a2 * I) / a1
    for _ in range(2, ORDER + 1):
        tn = _cheb_step_fused(L, t1, t0)
        Ts.append(tn)
        t0, t1 = t1, tn

    att4, out_nodes = _finalize(L, Ts, h, c)
    return out_nodes, att4.reshape(NF * N, N)
